# baseline (speedup 1.0000x reference)
import jax
import jax.numpy as jnp
from jax.experimental import pallas as pl

N, D = 16384, 128
TC_BLOCK_R = 8192
LD_BLOCK = TC_BLOCK_R // D


def _tc_body(x_ref, c_ref, lv_ref, b_ref, o_ref, ld_ref):
    c = c_ref[...]
    mask = c > 0.0
    s = jnp.exp(lv_ref[0, 0])
    o_ref[...] = jnp.where(mask, x_ref[...] * s + b_ref[0, 0],
                           x_ref[...])
    ones = jnp.full((D, 1), 1.0, dtype=jnp.float32)
    counts = jax.lax.dot_general(
        mask.astype(jnp.float32), ones,
        (((1,), (0,)), ((), ())),
        preferred_element_type=jnp.float32)
    ld_ref[...] = counts.reshape(LD_BLOCK, D) * lv_ref[0, 0]


_tc_transform = pl.pallas_call(
    _tc_body,
    grid=(N // TC_BLOCK_R,),
    in_specs=[
        pl.BlockSpec((TC_BLOCK_R, D), lambda i: (i, 0)),
        pl.BlockSpec((TC_BLOCK_R, D), lambda i: (i, 0)),
        pl.BlockSpec((1, 1), lambda i: (0, 0)),
        pl.BlockSpec((1, 1), lambda i: (0, 0)),
    ],
    out_specs=[
        pl.BlockSpec((TC_BLOCK_R, D), lambda i: (i, 0)),
        pl.BlockSpec((LD_BLOCK, D), lambda i: (i, 0)),
    ],
    out_shape=[
        jax.ShapeDtypeStruct((N, D), jnp.float32),
        jax.ShapeDtypeStruct((N // D, D), jnp.float32),
    ],
)


def kernel(inputs, context, log_scale, shift):
    lvs = log_scale.reshape(1, 1)
    bv = shift.reshape(1, 1)
    outputs, ld = _tc_transform(inputs, context, lvs, bv)
    return outputs, ld.reshape(N)
